# baseline (speedup 1.0000x reference)
import functools

import jax
import jax.numpy as jnp
from jax import lax
from jax.experimental import pallas as pl
from jax.experimental.pallas import tpu as pltpu
from jax.experimental.pallas import tpu_sc as plsc

D_MODEL = 64
IDX_W = 128
K_PER_CHUNK = 8
NUM_CORES = 2
NUM_SUBCORES = 16
NUM_WORKERS = NUM_CORES * NUM_SUBCORES


@functools.partial(jax.jit, static_argnums=(2,))
def _embed_flat(idx2d, weight, n_idx_rows):
    rows_per_w = n_idx_rows // NUM_WORKERS
    n_chunks = rows_per_w // K_PER_CHUNK
    rows_chunk = K_PER_CHUNK * IDX_W
    mesh = plsc.VectorSubcoreMesh(core_axis_name="c", subcore_axis_name="s")

    @functools.partial(
        pl.kernel,
        mesh=mesh,
        out_type=jax.ShapeDtypeStruct((n_idx_rows * IDX_W, D_MODEL),
                                      jnp.float32),
        scratch_types=[
            pltpu.VMEM((K_PER_CHUNK, IDX_W), jnp.int32),
            pltpu.VMEM((rows_chunk, D_MODEL), jnp.float32),
            pltpu.SemaphoreType.DMA,
        ],
        compiler_params=pltpu.CompilerParams(use_tc_tiling_on_sc=False),
    )
    def k(table_hbm, idx_hbm, out_hbm, idx_v, rows_v, sem):
        wid = lax.axis_index("s") * NUM_CORES + lax.axis_index("c")
        row0 = wid * rows_per_w

        def body(r, carry):
            base = row0 + r * K_PER_CHUNK
            pltpu.sync_copy(idx_hbm.at[pl.ds(base, K_PER_CHUNK)], idx_v)
            handles = [
                pltpu.async_copy(
                    table_hbm.at[idx_v.at[j]],
                    rows_v.at[pl.ds(j * IDX_W, IDX_W)],
                    sem,
                )
                for j in range(K_PER_CHUNK)
            ]
            for h in handles:
                h.wait()
            pltpu.sync_copy(
                rows_v, out_hbm.at[pl.ds(base * IDX_W, rows_chunk)])
            return carry

        lax.fori_loop(0, n_chunks, body, 0)

    return k(weight, idx2d)


def kernel(input_ids, weight):
    b, s = input_ids.shape
    total = b * s
    idx2d = input_ids.reshape(total // IDX_W, IDX_W).astype(jnp.int32)
    out = _embed_flat(idx2d, weight, total // IDX_W)
    return out.reshape(b, s, D_MODEL)


# speedup vs baseline: 1.0055x; 1.0055x over previous
import functools

import jax
import jax.numpy as jnp
from jax import lax
from jax.experimental import pallas as pl
from jax.experimental.pallas import tpu as pltpu
from jax.experimental.pallas import tpu_sc as plsc

D_MODEL = 64
IDX_W = 128
K_PER_CHUNK = 5
NBUF = 2
NUM_CORES = 2
NUM_SUBCORES = 16
NUM_WORKERS = NUM_CORES * NUM_SUBCORES


@functools.partial(jax.jit, static_argnums=(2,))
def _embed_flat(idx2d, weight, n_idx_rows):
    rows_per_w = n_idx_rows // NUM_WORKERS
    n_chunks = rows_per_w // K_PER_CHUNK
    rows_chunk = K_PER_CHUNK * IDX_W
    assert n_chunks % NBUF == 0 and n_chunks >= 2 * NBUF
    mesh = plsc.VectorSubcoreMesh(core_axis_name="c", subcore_axis_name="s")

    @functools.partial(
        pl.kernel,
        mesh=mesh,
        out_type=jax.ShapeDtypeStruct((n_idx_rows * IDX_W, D_MODEL),
                                      jnp.float32),
        scratch_types=[
            pltpu.VMEM((NBUF, K_PER_CHUNK, IDX_W), jnp.int32),
            pltpu.VMEM((NBUF, rows_chunk, D_MODEL), jnp.float32),
            [pltpu.SemaphoreType.DMA] * NBUF,
        ],
        compiler_params=pltpu.CompilerParams(use_tc_tiling_on_sc=False),
    )
    def k(table_hbm, idx_hbm, out_hbm, idx_v, rows_v, sems):
        wid = lax.axis_index("s") * NUM_CORES + lax.axis_index("c")
        row0 = wid * rows_per_w

        def stage(g, b):
            base = row0 + g * K_PER_CHUNK
            pltpu.sync_copy(idx_hbm.at[pl.ds(base, K_PER_CHUNK)],
                            idx_v.at[b])
            for j in range(K_PER_CHUNK):
                pltpu.async_copy(
                    table_hbm.at[idx_v.at[b, j]],
                    rows_v.at[b, pl.ds(j * IDX_W, IDX_W)],
                    sems[b],
                )

        def drain_store(g, b):
            for j in range(K_PER_CHUNK):
                pltpu.make_async_copy(
                    table_hbm.at[idx_v.at[b, j]],
                    rows_v.at[b, pl.ds(j * IDX_W, IDX_W)],
                    sems[b],
                ).wait()
            base = row0 + g * K_PER_CHUNK
            pltpu.sync_copy(rows_v.at[b],
                            out_hbm.at[pl.ds(base * IDX_W, rows_chunk)])

        for b in range(NBUF):
            stage(b, b)

        def body(g0, carry):
            for b in range(NBUF):
                g = g0 + b
                drain_store(g, b)
                stage(g + NBUF, b)
            return carry

        lax.fori_loop(0, (n_chunks - NBUF) // NBUF,
                      lambda i, c: body(i * NBUF, c), 0)
        for b in range(NBUF):
            drain_store(n_chunks - NBUF + b, b)

    return k(weight, idx2d)


def kernel(input_ids, weight):
    b, s = input_ids.shape
    total = b * s
    idx2d = input_ids.reshape(total // IDX_W, IDX_W).astype(jnp.int32)
    out = _embed_flat(idx2d, weight, total // IDX_W)
    return out.reshape(b, s, D_MODEL)
